# baseline (speedup 1.0000x reference)
import jax
import jax.numpy as jnp
from jax.experimental import pallas as pl
from jax.experimental.pallas import tpu as pltpu

_B = 4096
_T = 750
_WIN = 11
_SIGMA = 1.0
_E_THETA = 0.1
_E_G = 1.0
_E_ALPHA = 1.0
_TC = 128
_G = 6
_CB = 128
_AEH = 5376
_SW = 256
_NS_THRESH = 350.0
_JL = [j for j in range(_WIN) if j != 6]


def _loss_body(a_prev, a_cur, a_nxt, b_prev, b_cur, b_nxt, fb_ref,
               out_ref, g_ref, s4_ref, s3_ref, ae_ref):
    g = pl.program_id(0)

    lane = jax.lax.broadcasted_iota(jnp.int32, (1, _TC), 1) + g * _TC
    valid = lane < _T
    big = jnp.full((1, _TC), 1e9, jnp.float32)

    inv_two_sigma2 = jnp.float32(-0.5 / (_SIGMA * _SIGMA))
    dn = (((0,), (0,)), ((), ()))

    def fill_chunk(i, acc):
        r = pl.ds(i * _CB, _CB)
        s4_ref[r, 0:6] = a_prev[r, _TC - 6:_TC]
        s4_ref[r, 6:6 + _TC] = a_cur[r, :]
        s4_ref[r, 6 + _TC:16 + _TC] = a_nxt[r, 0:10]

        @pl.when(g == 0)
        def _():
            s4_ref[r, 0:6] = fb_ref[r, 0:6]

        @pl.when(g == _G - 1)
        def _():
            s4_ref[r, 116:121] = fb_ref[r, 6:11]

        d = a_cur[r, :] - b_cur[r, :]
        return acc + jnp.sum((d * d).reshape(_CB // 8, 8, _TC), axis=0)

    th = jax.lax.fori_loop(0, _B // _CB, fill_chunk,
                           jnp.zeros((8, _TC), jnp.float32))
    base = jnp.sum(th, axis=0, keepdims=True) * jnp.float32(_E_THETA)
    out_ref[...] = jnp.where(valid, base, 0.0).reshape(1, 1, _TC)

    s4a = s4_ref[:, :_TC]
    s4b = s4_ref[:, _TC:]
    g_aa = jax.lax.dot_general(s4a, s4a, dn, preferred_element_type=jnp.float32)
    g_ab = jax.lax.dot_general(s4a, s4b, dn, preferred_element_type=jnp.float32)
    g_bb = jax.lax.dot_general(s4b, s4b, dn, preferred_element_type=jnp.float32)
    g_ref[:_TC, :_TC] = g_aa
    g_ref[:_TC, _TC:] = g_ab
    g_ref[_TC:, :_TC] = g_ab.T
    g_ref[_TC:, _TC:] = g_bb

    rr = jax.lax.broadcasted_iota(jnp.int32, (_TC, _TC), 0)
    cc = jax.lax.broadcasted_iota(jnp.int32, (_TC, _TC), 1)
    eye = (rr == cc).astype(jnp.float32)

    def diag_at(row0, col0):
        blk = g_ref[row0:row0 + _TC, col0:col0 + _TC]
        return jnp.sum(blk * eye, axis=0, keepdims=True)

    cs_a = diag_at(0, 0)
    cs_b = diag_at(_TC, _TC)
    cs = jnp.concatenate([cs_a, cs_b], axis=1)
    cs6 = cs[:, 6:6 + _TC]
    min_ns = None
    for j in _JL:
        nsj = cs6 + cs[:, j:j + _TC] - 2.0 * diag_at(6, j)
        nsj = jnp.where(valid, nsj, big)
        min_ns = nsj if min_ns is None else jnp.minimum(min_ns, nsj)
    any_live = jnp.min(min_ns) < jnp.float32(_NS_THRESH)

    @pl.when(any_live)
    def _():
        def s3_chunk(i, _):
            r = pl.ds(i * _CB, _CB)
            s3_ref[r, 0:6] = b_prev[r, _TC - 6:_TC]
            s3_ref[r, 6:6 + _TC] = b_cur[r, :]
            s3_ref[r, 6 + _TC:16 + _TC] = b_nxt[r, 0:10]

            @pl.when(g == 0)
            def _():
                s3_ref[r, 0:6] = fb_ref[r, 11:17]

            @pl.when(g == _G - 1)
            def _():
                s3_ref[r, 116:121] = fb_ref[r, 17:22]

            return 0

        jax.lax.fori_loop(0, _B // _CB, s3_chunk, 0)

        def ns_chunk(i, carry):
            rows = pl.ds(i * _CB, _CB)
            ac = s4_ref[rows, 6:6 + _TC]
            new = [None] * len(_JL)
            for jj, j in enumerate(_JL):
                d = ac - s4_ref[rows, j:j + _TC]
                new[jj] = carry[jj] + jnp.sum(
                    (d * d).reshape(_CB // 8, 8, _TC), axis=0)
            return tuple(new)

        zeros = jnp.zeros((8, _TC), jnp.float32)
        ns_acc = jax.lax.fori_loop(0, _B // _CB, ns_chunk,
                                   (zeros,) * len(_JL))
        ens = [jnp.exp(inv_two_sigma2 *
                       jnp.where(valid,
                                 jnp.sum(ns_acc[jj], axis=0, keepdims=True),
                                 big))
               for jj in range(len(_JL))]

        def ae_fill(i, _):
            src = jax.lax.rem(i * _CB, jnp.int32(_B))
            ae_ref[pl.ds(i * _CB, _CB), :] = s4_ref[pl.ds(src, _CB),
                                                    6:6 + _TC]
            return 0

        jax.lax.fori_loop(0, _AEH // _CB, ae_fill, 0)

        def l1_chunk(i, tot):
            rows = pl.ds(i * _CB, _CB)
            s = jax.lax.rem(jnp.int32(11) * _CB * i, jnp.int32(_B))
            mw = ae_ref[pl.Slice(s, _CB, _WIN), :] - s4_ref[rows, 0:_TC]
            for j in range(1, _WIN):
                mw = jnp.maximum(
                    mw, ae_ref[pl.Slice(s + j, _CB, _WIN), :]
                    - s4_ref[rows, j:j + _TC])
            eg = jnp.exp(inv_two_sigma2 * jnp.float32(_E_G) * mw * mw)
            a2 = s3_ref[rows, 6:6 + _TC]
            acc = None
            for jj, j in enumerate(_JL):
                t = jnp.minimum(ens[jj], eg) * jnp.abs(
                    a2 - s3_ref[rows, j:j + _TC])
                acc = t if acc is None else acc + t
            return tot + jnp.sum(acc.reshape(_CB // 8, 8, _TC), axis=0)

        tot = jax.lax.fori_loop(0, _B // _CB, l1_chunk,
                                jnp.zeros((8, _TC), jnp.float32))
        part = jnp.sum(tot, axis=0, keepdims=True)
        out_ref[...] += jnp.where(valid, part, 0.0).reshape(1, 1, _TC)


def kernel(actioness, actioness_2):
    b = actioness.shape[0]
    fb = jnp.concatenate(
        [jnp.tile(actioness[:, 0], 6).reshape(b, 6),
         jnp.tile(actioness[:, -1], 6).reshape(b, 6)[:, 1:],
         jnp.tile(actioness_2[:, 0], 6).reshape(b, 6),
         jnp.tile(actioness_2[:, -1], 6).reshape(b, 6)[:, 1:],
         jnp.zeros((b, 10), jnp.float32)], axis=1)

    prev = pl.BlockSpec((_B, _TC), lambda i: (0, jnp.maximum(i - 1, 0)))
    cur = pl.BlockSpec((_B, _TC), lambda i: (0, i))
    nxt = pl.BlockSpec((_B, _TC), lambda i: (0, jnp.minimum(i + 1, _G - 1)))
    fb_spec = pl.BlockSpec((_B, 32), lambda i: (0, 0))

    partials = pl.pallas_call(
        _loss_body,
        grid=(_G,),
        in_specs=[prev, cur, nxt, prev, cur, nxt, fb_spec],
        out_specs=pl.BlockSpec((1, 1, _TC), lambda i: (i, 0, 0)),
        out_shape=jax.ShapeDtypeStruct((_G, 1, _TC), jnp.float32),
        scratch_shapes=[
            pltpu.VMEM((2 * _TC, 2 * _TC), jnp.float32),
            pltpu.VMEM((_B, _SW), jnp.float32),
            pltpu.VMEM((_B, _SW), jnp.float32),
            pltpu.VMEM((_AEH, _TC), jnp.float32),
        ],
        compiler_params=pltpu.CompilerParams(
            dimension_semantics=("parallel",),
            vmem_limit_bytes=48 * 1024 * 1024,
        ),
        name="act_eloss_v3",
    )(actioness, actioness, actioness, actioness_2, actioness_2, actioness_2,
      fb)

    return jnp.float32(_E_ALPHA / _B) * jnp.sum(partials)
